# baseline (device time: 146717 ns/iter reference)
import jax
import jax.numpy as jnp
from jax import lax
from jax.experimental import pallas as pl
from jax.experimental.pallas import tpu as pltpu

N_DEV = 16
SQ = 1024
SKV = 1024
D = 1024
HQ = 8
DH = 128
BLK = 64
CHUNK = SQ // N_DEV
SCALE = 0.08838834764831843


def kernel(x, Wq, K_ext, V_ext, Wo):
    def body(x_ref, wq_hbm, k_ref, v_ref, wo_hbm, out_ref,
             wq_ref, wo_ref, q_ref, partial_ref, rs_ref,
             load_sems, send_a, recv_a, send_b, recv_b):
        me = lax.axis_index("i")

        wq_cp = pltpu.make_async_copy(
            wq_hbm.at[:, pl.ds(me * D, D)], wq_ref, load_sems.at[0])
        wq_cp.start()
        wo_cp = pltpu.make_async_copy(
            wo_hbm.at[pl.ds(me * D, D), :], wo_ref, load_sems.at[1])
        wo_cp.start()

        barrier = pltpu.get_barrier_semaphore()
        for off in range(1, N_DEV):
            peer = lax.rem(me + off, N_DEV)
            pl.semaphore_signal(barrier, inc=1, device_id=(peer,),
                                device_id_type=pl.DeviceIdType.MESH)
        pl.semaphore_wait(barrier, N_DEV - 1)

        x_bf = x_ref[0].astype(jnp.bfloat16)
        wq_cp.wait()
        q_all = jnp.dot(x_bf, wq_ref[...].astype(jnp.bfloat16),
                        preferred_element_type=jnp.float32)
        q_ref[...] = q_all.astype(jnp.bfloat16)
        k_bf = k_ref[0].astype(jnp.bfloat16)
        v_bf = v_ref[0].astype(jnp.bfloat16)
        wo_cp.wait()
        wo_bf = wo_ref[...].astype(jnp.bfloat16)
        kvblk = lax.broadcasted_iota(jnp.int32, (1, SKV), 1) // BLK

        def step(i, carry):
            c = lax.rem(me + 1 + i, N_DEV)
            r0 = c * CHUNK
            qc = q_ref[pl.ds(r0, CHUNK), :]
            keep = kvblk <= c
            parts = []
            for h in range(HQ):
                qh = lax.slice_in_dim(qc, h * DH, (h + 1) * DH, axis=1)
                s = lax.dot_general(qh, k_bf[:, h, :],
                                    (((1,), (1,)), ((), ())),
                                    preferred_element_type=jnp.float32)
                s = jnp.where(keep, s * SCALE, -1e9)
                m = jnp.max(s, axis=1, keepdims=True)
                w = jnp.exp(s - m)
                w = w / jnp.sum(w, axis=1, keepdims=True)
                parts.append(jnp.dot(w.astype(jnp.bfloat16), v_bf[:, h, :],
                                     preferred_element_type=jnp.float32))
            ctxc = jnp.concatenate(parts, axis=1).astype(jnp.bfloat16)
            pch = jnp.dot(ctxc, wo_bf,
                          preferred_element_type=jnp.float32)
            pch = pch.astype(jnp.bfloat16)
            partial_ref[pl.ds(r0, CHUNK), :] = pch

            @pl.when(c != me)
            def _():
                pltpu.make_async_remote_copy(
                    src_ref=partial_ref.at[pl.ds(r0, CHUNK), :],
                    dst_ref=rs_ref.at[me],
                    send_sem=send_a, recv_sem=recv_a,
                    device_id=(c,),
                    device_id_type=pl.DeviceIdType.MESH).start()

            @pl.when(c == me)
            def _():
                rs_ref[pl.ds(me, 1)] = pch[None]

            return carry

        lax.fori_loop(0, N_DEV, step, 0)

        for _ in range(N_DEV - 1):
            pltpu.make_async_remote_copy(
                src_ref=rs_ref.at[0], dst_ref=rs_ref.at[0],
                send_sem=send_a, recv_sem=recv_a,
                device_id=(me,), device_id_type=pl.DeviceIdType.MESH,
            ).wait_recv()

        my_rows = pl.ds(me * CHUNK, CHUNK)
        reduced = jnp.sum(rs_ref[...].astype(jnp.float32), axis=0)
        out_ref[0, my_rows, :] = reduced.astype(jnp.bfloat16)

        sends_b = []
        for off in range(1, N_DEV):
            peer = lax.rem(me + off, N_DEV)
            rdma = pltpu.make_async_remote_copy(
                src_ref=out_ref.at[0, my_rows, :],
                dst_ref=out_ref.at[0, my_rows, :],
                send_sem=send_b, recv_sem=recv_b,
                device_id=(peer,), device_id_type=pl.DeviceIdType.MESH)
            rdma.start()
            sends_b.append(rdma)

        for _ in range(N_DEV - 1):
            pltpu.make_async_remote_copy(
                src_ref=partial_ref.at[pl.ds(0, CHUNK), :],
                dst_ref=rs_ref.at[0],
                send_sem=send_a, recv_sem=recv_a,
                device_id=(me,), device_id_type=pl.DeviceIdType.MESH,
            ).wait_send()

        for _ in range(N_DEV - 1):
            pltpu.make_async_remote_copy(
                src_ref=out_ref.at[0, pl.ds(0, CHUNK), :],
                dst_ref=out_ref.at[0, pl.ds(0, CHUNK), :],
                send_sem=send_b, recv_sem=recv_b,
                device_id=(me,), device_id_type=pl.DeviceIdType.MESH,
            ).wait_recv()
        for rdma in sends_b:
            rdma.wait_send()

    return pl.pallas_call(
        body,
        out_shape=jax.ShapeDtypeStruct((1, SQ, D), jnp.bfloat16),
        in_specs=[
            pl.BlockSpec(memory_space=pltpu.VMEM),
            pl.BlockSpec(memory_space=pltpu.MemorySpace.HBM),
            pl.BlockSpec(memory_space=pltpu.VMEM),
            pl.BlockSpec(memory_space=pltpu.VMEM),
            pl.BlockSpec(memory_space=pltpu.MemorySpace.HBM),
        ],
        out_specs=pl.BlockSpec(memory_space=pltpu.VMEM),
        scratch_shapes=[
            pltpu.VMEM((D, D), jnp.float32),
            pltpu.VMEM((D, D), jnp.float32),
            pltpu.VMEM((SQ, HQ * DH), jnp.bfloat16),
            pltpu.VMEM((SQ, D), jnp.bfloat16),
            pltpu.VMEM((N_DEV, CHUNK, D), jnp.bfloat16),
            pltpu.SemaphoreType.DMA((2,)),
            pltpu.SemaphoreType.DMA,
            pltpu.SemaphoreType.DMA,
            pltpu.SemaphoreType.DMA,
            pltpu.SemaphoreType.DMA,
        ],
        compiler_params=pltpu.CompilerParams(
            collective_id=0, vmem_limit_bytes=100 * 1024 * 1024),
    )(x, Wq, K_ext, V_ext, Wo)


# device time: 79041 ns/iter; 1.8562x vs baseline; 1.8562x over previous
import jax
import jax.numpy as jnp
from jax import lax
from jax.experimental import pallas as pl
from jax.experimental.pallas import tpu as pltpu

N_DEV = 16
SQ = 1024
SKV = 1024
D = 1024
HQ = 8
DH = 128
BLK = 64
CHUNK = SQ // N_DEV
SCALE = 0.08838834764831843


def kernel(x, Wq, K_ext, V_ext, Wo):
    def body(x_ref, wq_hbm, k_ref, v_ref, wo_hbm, out_ref,
             wq_ref, wo_ref, q_ref, partial_ref, rs_ref,
             load_sems, send_a, recv_a, send_b, recv_b):
        me = lax.axis_index("i")

        wq_cp = pltpu.make_async_copy(
            wq_hbm.at[:, pl.ds(me * D, D)], wq_ref, load_sems.at[0])
        wq_cp.start()
        wo_cp = pltpu.make_async_copy(
            wo_hbm.at[pl.ds(me * D, D), :], wo_ref, load_sems.at[1])
        wo_cp.start()

        barrier = pltpu.get_barrier_semaphore()
        for off in range(1, N_DEV):
            peer = lax.rem(me + off, N_DEV)
            pl.semaphore_signal(barrier, inc=1, device_id=(peer,),
                                device_id_type=pl.DeviceIdType.MESH)
        pl.semaphore_wait(barrier, N_DEV - 1)

        x_bf = x_ref[0].astype(jnp.bfloat16)
        wq_cp.wait()
        q_all = jnp.dot(x_bf, wq_ref[...].astype(jnp.bfloat16),
                        preferred_element_type=jnp.float32)
        q_ref[...] = q_all.astype(jnp.bfloat16)
        k_bf = k_ref[0].astype(jnp.bfloat16)
        v_bf = v_ref[0].astype(jnp.bfloat16)
        wo_cp.wait()
        wo_bf = wo_ref[...].astype(jnp.bfloat16)

        GROUP = 256
        NG = SQ // GROUP
        for g in range(NG):
            kl = (g + 1) * GROUP
            r0 = g * GROUP
            qg = q_ref[r0:r0 + GROUP, :]
            qb = r0 // BLK + lax.broadcasted_iota(jnp.int32, (GROUP, kl), 0) // BLK
            kb = lax.broadcasted_iota(jnp.int32, (GROUP, kl), 1) // BLK
            keep = kb <= qb
            parts = []
            for h in range(HQ):
                qh = qg[:, h * DH:(h + 1) * DH]
                s = lax.dot_general(qh, k_bf[:kl, h, :],
                                    (((1,), (1,)), ((), ())),
                                    preferred_element_type=jnp.float32)
                s = jnp.where(keep, s * SCALE, -1e9)
                m = jnp.max(s, axis=1, keepdims=True)
                w = jnp.exp(s - m)
                w = w / jnp.sum(w, axis=1, keepdims=True)
                parts.append(jnp.dot(w.astype(jnp.bfloat16), v_bf[:kl, h, :],
                                     preferred_element_type=jnp.float32))
            ctxg = jnp.concatenate(parts, axis=1).astype(jnp.bfloat16)
            pch = jnp.dot(ctxg, wo_bf,
                          preferred_element_type=jnp.float32)
            pch = pch.astype(jnp.bfloat16)
            partial_ref[r0:r0 + GROUP, :] = pch

            for j in range(GROUP // CHUNK):
                c = g * (GROUP // CHUNK) + j
                c0 = j * CHUNK

                @pl.when(c != me)
                def _(c=c, c0=c0):
                    pltpu.make_async_remote_copy(
                        src_ref=partial_ref.at[
                            pl.ds(c * CHUNK, CHUNK), :],
                        dst_ref=rs_ref.at[me],
                        send_sem=send_a, recv_sem=recv_a,
                        device_id=(c,),
                        device_id_type=pl.DeviceIdType.MESH).start()

                @pl.when(c == me)
                def _(c0=c0):
                    rs_ref[pl.ds(me, 1)] = pch[None, c0:c0 + CHUNK, :]

        for _ in range(N_DEV - 1):
            pltpu.make_async_remote_copy(
                src_ref=rs_ref.at[0], dst_ref=rs_ref.at[0],
                send_sem=send_a, recv_sem=recv_a,
                device_id=(me,), device_id_type=pl.DeviceIdType.MESH,
            ).wait_recv()

        my_rows = pl.ds(me * CHUNK, CHUNK)
        reduced = jnp.sum(rs_ref[...].astype(jnp.float32), axis=0)
        out_ref[0, my_rows, :] = reduced.astype(jnp.bfloat16)

        sends_b = []
        for off in range(1, N_DEV):
            peer = lax.rem(me + off, N_DEV)
            rdma = pltpu.make_async_remote_copy(
                src_ref=out_ref.at[0, my_rows, :],
                dst_ref=out_ref.at[0, my_rows, :],
                send_sem=send_b, recv_sem=recv_b,
                device_id=(peer,), device_id_type=pl.DeviceIdType.MESH)
            rdma.start()
            sends_b.append(rdma)

        for _ in range(N_DEV - 1):
            pltpu.make_async_remote_copy(
                src_ref=partial_ref.at[pl.ds(0, CHUNK), :],
                dst_ref=rs_ref.at[0],
                send_sem=send_a, recv_sem=recv_a,
                device_id=(me,), device_id_type=pl.DeviceIdType.MESH,
            ).wait_send()

        for _ in range(N_DEV - 1):
            pltpu.make_async_remote_copy(
                src_ref=out_ref.at[0, pl.ds(0, CHUNK), :],
                dst_ref=out_ref.at[0, pl.ds(0, CHUNK), :],
                send_sem=send_b, recv_sem=recv_b,
                device_id=(me,), device_id_type=pl.DeviceIdType.MESH,
            ).wait_recv()
        for rdma in sends_b:
            rdma.wait_send()

    return pl.pallas_call(
        body,
        out_shape=jax.ShapeDtypeStruct((1, SQ, D), jnp.bfloat16),
        in_specs=[
            pl.BlockSpec(memory_space=pltpu.VMEM),
            pl.BlockSpec(memory_space=pltpu.MemorySpace.HBM),
            pl.BlockSpec(memory_space=pltpu.VMEM),
            pl.BlockSpec(memory_space=pltpu.VMEM),
            pl.BlockSpec(memory_space=pltpu.MemorySpace.HBM),
        ],
        out_specs=pl.BlockSpec(memory_space=pltpu.VMEM),
        scratch_shapes=[
            pltpu.VMEM((D, D), jnp.float32),
            pltpu.VMEM((D, D), jnp.float32),
            pltpu.VMEM((SQ, HQ * DH), jnp.bfloat16),
            pltpu.VMEM((SQ, D), jnp.bfloat16),
            pltpu.VMEM((N_DEV, CHUNK, D), jnp.bfloat16),
            pltpu.SemaphoreType.DMA((2,)),
            pltpu.SemaphoreType.DMA,
            pltpu.SemaphoreType.DMA,
            pltpu.SemaphoreType.DMA,
            pltpu.SemaphoreType.DMA,
        ],
        compiler_params=pltpu.CompilerParams(
            collective_id=0, vmem_limit_bytes=100 * 1024 * 1024),
    )(x, Wq, K_ext, V_ext, Wo)


# device time: 64581 ns/iter; 2.2718x vs baseline; 1.2239x over previous
import jax
import jax.numpy as jnp
from jax import lax
from jax.experimental import pallas as pl
from jax.experimental.pallas import tpu as pltpu

N_DEV = 16
SQ = 1024
SKV = 1024
D = 1024
HQ = 8
DH = 128
BLK = 64
GROUP = 256
NG = SQ // GROUP
SLIV = GROUP // N_DEV
SCALE = 0.08838834764831843


def kernel(x, Wq, K_ext, V_ext, Wo):
    def body(x_ref, wq_hbm, k_ref, v_ref, wo_hbm, out_ref,
             wq_ref, wo_ref, q_ref, partial_ref, rs_ref,
             load_sems, send_a, recv_a, send_b, recv_b):
        me = lax.axis_index("i")

        wq_cp = pltpu.make_async_copy(
            wq_hbm.at[:, pl.ds(me * D, D)], wq_ref, load_sems.at[0])
        wq_cp.start()
        wo_cp = pltpu.make_async_copy(
            wo_hbm.at[pl.ds(me * D, D), :], wo_ref, load_sems.at[1])
        wo_cp.start()

        barrier = pltpu.get_barrier_semaphore()
        for off in range(1, N_DEV):
            peer = lax.rem(me + off, N_DEV)
            pl.semaphore_signal(barrier, inc=1, device_id=(peer,),
                                device_id_type=pl.DeviceIdType.MESH)
        pl.semaphore_wait(barrier, N_DEV - 1)

        x_bf = x_ref[0].astype(jnp.bfloat16)
        wq_cp.wait()
        q_all = jnp.dot(x_bf, wq_ref[...].astype(jnp.bfloat16),
                        preferred_element_type=jnp.float32)
        q_ref[...] = (q_all * SCALE).astype(jnp.bfloat16)
        k_bf = k_ref[0].astype(jnp.bfloat16)
        v_bf = v_ref[0].astype(jnp.bfloat16)
        wo_cp.wait()
        wo_bf = wo_ref[...].astype(jnp.bfloat16)

        rowb = lax.broadcasted_iota(jnp.int32, (GROUP, GROUP), 0) // BLK
        colb = lax.broadcasted_iota(jnp.int32, (GROUP, GROUP), 1) // BLK
        diag_bias = jnp.where(colb <= rowb, 0.0, -1e9).astype(jnp.float32)

        for g in range(NG):
            kl = (g + 1) * GROUP
            r0 = g * GROUP
            qg = q_ref[r0:r0 + GROUP, :]
            if g > 0:
                bias = jnp.concatenate(
                    [jnp.zeros((GROUP, r0), jnp.float32), diag_bias], axis=1)
            else:
                bias = diag_bias
            parts = []
            for h in range(HQ):
                qh = qg[:, h * DH:(h + 1) * DH]
                s = lax.dot_general(qh, k_bf[:kl, h, :],
                                    (((1,), (1,)), ((), ())),
                                    preferred_element_type=jnp.float32)
                w = jnp.exp(s + bias)
                recip = 1.0 / jnp.sum(w, axis=1, keepdims=True)
                u = jnp.dot(w.astype(jnp.bfloat16), v_bf[:kl, h, :],
                            preferred_element_type=jnp.float32)
                parts.append(u * recip)
            ctxg = jnp.concatenate(parts, axis=1).astype(jnp.bfloat16)
            pch = jnp.dot(ctxg, wo_bf,
                          preferred_element_type=jnp.float32)
            partial_ref[r0:r0 + GROUP, :] = pch.astype(jnp.bfloat16)

            my_sliv = pl.ds(r0 + me * SLIV, SLIV)
            rs_ref[0, g] = partial_ref[my_sliv, :]
            for off in range(1, N_DEV):
                peer = lax.rem(me + off, N_DEV)
                pltpu.make_async_remote_copy(
                    src_ref=partial_ref.at[
                        pl.ds(r0 + peer * SLIV, SLIV), :],
                    dst_ref=rs_ref.at[N_DEV - off, g],
                    send_sem=send_a, recv_sem=recv_a,
                    device_id=(peer,),
                    device_id_type=pl.DeviceIdType.MESH).start()

        for _ in range(NG * (N_DEV - 1)):
            pltpu.make_async_remote_copy(
                src_ref=rs_ref.at[1, 0], dst_ref=rs_ref.at[1, 0],
                send_sem=send_a, recv_sem=recv_a,
                device_id=(me,), device_id_type=pl.DeviceIdType.MESH,
            ).wait_recv()

        reduced = jnp.sum(rs_ref[...].astype(jnp.float32), axis=0)
        red_bf = reduced.astype(jnp.bfloat16)
        for g in range(NG):
            out_ref[0, pl.ds(g * GROUP + me * SLIV, SLIV), :] = red_bf[g]

        for off in range(1, N_DEV):
            peer = lax.rem(me + off, N_DEV)
            for g in range(NG):
                pltpu.make_async_remote_copy(
                    src_ref=out_ref.at[0, pl.ds(g * GROUP + me * SLIV, SLIV), :],
                    dst_ref=out_ref.at[0, pl.ds(g * GROUP + me * SLIV, SLIV), :],
                    send_sem=send_b, recv_sem=recv_b,
                    device_id=(peer,),
                    device_id_type=pl.DeviceIdType.MESH).start()

        for _ in range(NG * (N_DEV - 1)):
            pltpu.make_async_remote_copy(
                src_ref=partial_ref.at[pl.ds(0, SLIV), :],
                dst_ref=rs_ref.at[1, 0],
                send_sem=send_a, recv_sem=recv_a,
                device_id=(me,), device_id_type=pl.DeviceIdType.MESH,
            ).wait_send()

        for _ in range(NG * (N_DEV - 1)):
            pltpu.make_async_remote_copy(
                src_ref=out_ref.at[0, pl.ds(0, SLIV), :],
                dst_ref=out_ref.at[0, pl.ds(0, SLIV), :],
                send_sem=send_b, recv_sem=recv_b,
                device_id=(me,), device_id_type=pl.DeviceIdType.MESH,
            ).wait_recv()
        for _ in range(NG * (N_DEV - 1)):
            pltpu.make_async_remote_copy(
                src_ref=out_ref.at[0, pl.ds(0, SLIV), :],
                dst_ref=out_ref.at[0, pl.ds(0, SLIV), :],
                send_sem=send_b, recv_sem=recv_b,
                device_id=(me,), device_id_type=pl.DeviceIdType.MESH,
            ).wait_send()

    return pl.pallas_call(
        body,
        out_shape=jax.ShapeDtypeStruct((1, SQ, D), jnp.bfloat16),
        in_specs=[
            pl.BlockSpec(memory_space=pltpu.VMEM),
            pl.BlockSpec(memory_space=pltpu.MemorySpace.HBM),
            pl.BlockSpec(memory_space=pltpu.VMEM),
            pl.BlockSpec(memory_space=pltpu.VMEM),
            pl.BlockSpec(memory_space=pltpu.MemorySpace.HBM),
        ],
        out_specs=pl.BlockSpec(memory_space=pltpu.VMEM),
        scratch_shapes=[
            pltpu.VMEM((D, D), jnp.float32),
            pltpu.VMEM((D, D), jnp.float32),
            pltpu.VMEM((SQ, HQ * DH), jnp.bfloat16),
            pltpu.VMEM((SQ, D), jnp.bfloat16),
            pltpu.VMEM((N_DEV, NG, SLIV, D), jnp.bfloat16),
            pltpu.SemaphoreType.DMA((2,)),
            pltpu.SemaphoreType.DMA,
            pltpu.SemaphoreType.DMA,
            pltpu.SemaphoreType.DMA,
            pltpu.SemaphoreType.DMA,
        ],
        compiler_params=pltpu.CompilerParams(
            collective_id=0, vmem_limit_bytes=100 * 1024 * 1024),
    )(x, Wq, K_ext, V_ext, Wo)
